# baseline (speedup 1.0000x reference)
import functools

import jax
import jax.numpy as jnp
from jax import lax
from jax.experimental import pallas as pl
from jax.experimental.pallas import tpu as pltpu
from jax.experimental.pallas import tpu_sc as plsc

BATCH = 16384
STUDENT_DIM = 32
ITEM_DIM = 32
DISC_DIM = 8

NC = 2
NS = 16
NW = NC * NS
B_PER_W = BATCH // NW
CHUNK = 128
N_CHUNK = B_PER_W // CHUNK


def _build():
    mesh = plsc.VectorSubcoreMesh(core_axis_name="c", subcore_axis_name="s")

    @functools.partial(
        pl.kernel,
        mesh=mesh,
        compiler_params=pltpu.CompilerParams(use_tc_tiling_on_sc=False),
        out_type=(
            jax.ShapeDtypeStruct((BATCH, STUDENT_DIM), jnp.float32),
            jax.ShapeDtypeStruct((BATCH, ITEM_DIM), jnp.float32),
            jax.ShapeDtypeStruct((BATCH, DISC_DIM), jnp.float32),
        ),
        scratch_types=[
            pltpu.VMEM((N_CHUNK, CHUNK), jnp.int32),
            pltpu.VMEM((N_CHUNK, CHUNK), jnp.int32),
            pltpu.VMEM((B_PER_W, STUDENT_DIM), jnp.float32),
            pltpu.VMEM((B_PER_W, ITEM_DIM), jnp.float32),
            pltpu.VMEM((B_PER_W, DISC_DIM), jnp.float32),
            pltpu.SemaphoreType.DMA,
            pltpu.SemaphoreType.DMA,
            pltpu.SemaphoreType.DMA,
        ],
    )
    def emb_kernel(stu_idx_hbm, exer_idx_hbm, stu_tab, item_tab, disc_tab,
                   stu_out, item_out, disc_out,
                   sidx_v, eidx_v, srows_v, irows_v, drows_v,
                   sem_s, sem_i, sem_d):
        wid = lax.axis_index("s") * NC + lax.axis_index("c")
        base = wid * B_PER_W
        row0 = wid * N_CHUNK
        pltpu.sync_copy(stu_idx_hbm.at[pl.ds(row0, N_CHUNK)], sidx_v)
        pltpu.sync_copy(exer_idx_hbm.at[pl.ds(row0, N_CHUNK)], eidx_v)
        waits = []
        for j in range(N_CHUNK):
            waits.append(pltpu.async_copy(
                stu_tab.at[sidx_v.at[j]],
                srows_v.at[pl.ds(j * CHUNK, CHUNK)], sem_s))
        for j in range(N_CHUNK):
            waits.append(pltpu.async_copy(
                item_tab.at[eidx_v.at[j]],
                irows_v.at[pl.ds(j * CHUNK, CHUNK)], sem_i))
        for j in range(N_CHUNK):
            waits.append(pltpu.async_copy(
                disc_tab.at[eidx_v.at[j]],
                drows_v.at[pl.ds(j * CHUNK, CHUNK)], sem_d))
        for w in waits[:N_CHUNK]:
            w.wait()
        pltpu.sync_copy(srows_v, stu_out.at[pl.ds(base, B_PER_W)])
        for w in waits[N_CHUNK:2 * N_CHUNK]:
            w.wait()
        pltpu.sync_copy(irows_v, item_out.at[pl.ds(base, B_PER_W)])
        for w in waits[2 * N_CHUNK:]:
            w.wait()
        pltpu.sync_copy(drows_v, disc_out.at[pl.ds(base, B_PER_W)])

    return emb_kernel


_EMB_KERNEL = _build()


def kernel(stu_idx, exer_idx, student_table, item_table, disc_table):
    stu_idx2 = stu_idx.astype(jnp.int32).reshape(NW * N_CHUNK, CHUNK)
    exer_idx2 = exer_idx.astype(jnp.int32).reshape(NW * N_CHUNK, CHUNK)
    return _EMB_KERNEL(stu_idx2, exer_idx2,
                       student_table, item_table, disc_table)


# speedup vs baseline: 1.7069x; 1.7069x over previous
import functools

import jax
import jax.numpy as jnp
from jax import lax
from jax.experimental import pallas as pl
from jax.experimental.pallas import tpu as pltpu
from jax.experimental.pallas import tpu_sc as plsc

BATCH = 16384
SDIM = 32
IDIM = 32
DDIM = 8

NC = 2
NS = 16
NW = NC * NS
B_PER_W = BATCH // NW
N_LT = B_PER_W // 128
GROUPS_PER_LT = 128 // 16


def _build():
    mesh = plsc.VectorSubcoreMesh(core_axis_name="c", subcore_axis_name="s")

    @functools.partial(
        pl.kernel,
        mesh=mesh,
        compiler_params=pltpu.CompilerParams(needs_layout_passes=False),
        out_type=(
            jax.ShapeDtypeStruct((SDIM, BATCH), jnp.float32),
            jax.ShapeDtypeStruct((IDIM, BATCH), jnp.float32),
            jax.ShapeDtypeStruct((DDIM, BATCH), jnp.float32),
        ),
        scratch_types=[
            pltpu.VMEM((B_PER_W,), jnp.int32),
            pltpu.VMEM((B_PER_W,), jnp.int32),
            pltpu.VMEM((2, SDIM, 128), jnp.float32),
            pltpu.VMEM((2, IDIM, 128), jnp.float32),
            pltpu.VMEM((2, DDIM, 128), jnp.float32),
            pltpu.VMEM((SDIM, 128), jnp.float32),
            pltpu.VMEM((IDIM, 128), jnp.float32),
            pltpu.VMEM((DDIM, 128), jnp.float32),
            pltpu.SemaphoreType.DMA,
            pltpu.SemaphoreType.DMA,
            pltpu.SemaphoreType.DMA,
        ],
    )
    def emb_kernel(stu_idx_hbm, exer_idx_hbm, stu_tab, item_tab, disc_tab,
                   stu_out, item_out, disc_out,
                   sidx_v, eidx_v, sfetch, ifetch, dfetch,
                   sstage, istage, dstage,
                   sem_s, sem_i, sem_d):
        wid = lax.axis_index("s") * NC + lax.axis_index("c")
        base = wid * B_PER_W
        pltpu.sync_copy(stu_idx_hbm.at[pl.ds(base, B_PER_W)], sidx_v)
        pltpu.sync_copy(exer_idx_hbm.at[pl.ds(base, B_PER_W)], eidx_v)

        rows = lax.iota(jnp.int32, 16)

        def fetch(cvec_s, cvec_e, t, slot):
            cs = cvec_s[t]
            ce = cvec_e[t]
            w_s = pl.multiple_of((cs >> 7) << 7, 128)
            w_e = pl.multiple_of((ce >> 7) << 7, 128)
            a = pltpu.async_copy(
                stu_tab.at[:, pl.ds(w_s, 128)], sfetch.at[slot], sem_s)
            b = pltpu.async_copy(
                item_tab.at[:, pl.ds(w_e, 128)], ifetch.at[slot], sem_i)
            c = pltpu.async_copy(
                disc_tab.at[:, pl.ds(w_e, 128)], dfetch.at[slot], sem_d)
            return (a, b, c)

        def extract(cvec_s, cvec_e, t, slot, lane_i):
            qs = jnp.full((16,), cvec_s[t] & 127, jnp.int32)
            qe = jnp.full((16,), cvec_e[t] & 127, jnp.int32)
            lane = jnp.full((16,), lane_i, jnp.int32)
            v0 = plsc.load_gather(sfetch.at[slot], [rows, qs])
            v1 = plsc.load_gather(sfetch.at[slot], [rows + 16, qs])
            plsc.store_scatter(sstage, [rows, lane], v0)
            plsc.store_scatter(sstage, [rows + 16, lane], v1)
            w0 = plsc.load_gather(ifetch.at[slot], [rows, qe])
            w1 = plsc.load_gather(ifetch.at[slot], [rows + 16, qe])
            plsc.store_scatter(istage, [rows, lane], w0)
            plsc.store_scatter(istage, [rows + 16, lane], w1)
            mask8 = rows < 8
            u0 = plsc.load_gather(dfetch.at[slot], [rows & 7, qe], mask=mask8)
            plsc.store_scatter(dstage, [rows & 7, lane], u0, mask=mask8)

        for jj in range(N_LT):
            def body(g, _):
                goff = jj * 128 + g * 16
                cvec_s = sidx_v[pl.ds(goff, 16)]
                cvec_e = eidx_v[pl.ds(goff, 16)]
                waits = fetch(cvec_s, cvec_e, 0, 0)
                for t in range(16):
                    nxt = None
                    if t < 15:
                        nxt = fetch(cvec_s, cvec_e, t + 1, (t + 1) % 2)
                    for w in waits:
                        w.wait()
                    extract(cvec_s, cvec_e, t, t % 2, g * 16 + t)
                    waits = nxt
                return _

            lax.fori_loop(0, GROUPS_PER_LT, body, 0)
            lane0 = pl.multiple_of((wid * N_LT + jj) * 128, 128)
            pltpu.sync_copy(sstage, stu_out.at[:, pl.ds(lane0, 128)])
            pltpu.sync_copy(istage, item_out.at[:, pl.ds(lane0, 128)])
            pltpu.sync_copy(dstage, disc_out.at[:, pl.ds(lane0, 128)])

    return emb_kernel


_EMB_KERNEL = _build()


def kernel(stu_idx, exer_idx, student_table, item_table, disc_table):
    outs = _EMB_KERNEL(stu_idx.astype(jnp.int32), exer_idx.astype(jnp.int32),
                       student_table.T, item_table.T, disc_table.T)
    return (outs[0].T, outs[1].T, outs[2].T)
